# baseline (speedup 1.0000x reference)
import functools

import jax
import jax.numpy as jnp
from jax import lax
from jax.experimental import pallas as pl
from jax.experimental.pallas import tpu as pltpu
from jax.experimental.pallas import tpu_sc as plsc

K = 4096
NC, NS = 2, 16
NW = NC * NS
A4K = 4 * K
CHUNK = 8192

_mesh = plsc.VectorSubcoreMesh(
    core_axis_name="c", subcore_axis_name="s", num_cores=NC, num_subcores=NS)
_params = pltpu.CompilerParams(needs_layout_passes=False)

_Z16F = functools.partial(jnp.zeros, (16,), jnp.float32)
_Z16I = functools.partial(jnp.zeros, (16,), jnp.int32)
_SIGN = -2147483648
_FLIP = 0x7FFFFFFF


def _zero_ref(ref, nwords, dtype=jnp.float32):
    z = jnp.zeros((16,), dtype)

    @plsc.parallel_loop(0, nwords, step=16, unroll=8)
    def body(off):
        ref[pl.ds(off, 16)] = z


@functools.lru_cache(maxsize=None)
def _build(n):
    assert n % NW == 0
    per_w = n // NW
    chunk = min(CHUNK, per_w)
    assert per_w % chunk == 0 and chunk % 64 == 0
    nchunk = per_w // chunk
    assert nchunk % 2 == 0, "ring pipeline processes chunks in pairs"
    red_w = A4K // NS

    @functools.partial(
        pl.kernel,
        out_type=jax.ShapeDtypeStruct((NC, A4K), jnp.float32),
        mesh=_mesh,
        scratch_types=[
            [pltpu.VMEM((K,), jnp.float32)] * 4,
            pltpu.VMEM((2, chunk), jnp.float32),
            pltpu.VMEM((2, chunk), jnp.float32),
            pltpu.VMEM((2, chunk), jnp.int32),
            pltpu.VMEM_SHARED((NS, A4K), jnp.float32),
            pltpu.VMEM((red_w,), jnp.float32),
            pltpu.VMEM((2, red_w), jnp.float32),
            pltpu.SemaphoreType.DMA,
            pltpu.SemaphoreType.DMA,
        ],
        compiler_params=_params,
    )
    def _accumulate(yp_hbm, yt_hbm, bs_hbm, out_hbm,
                    accs, ypb, ytb, bsb, shared, racc, rbuf, sem_in, sem_red):
        cid = lax.axis_index("c")
        sid = lax.axis_index("s")
        wid = cid * NS + sid
        base = wid * per_w
        for a in accs:
            _zero_ref(a, K)

        def start(c, buf):
            off = base + c * chunk
            return (
                pltpu.async_copy(yp_hbm.at[pl.ds(off, chunk)], ypb.at[buf], sem_in),
                pltpu.async_copy(yt_hbm.at[pl.ds(off, chunk)], ytb.at[buf], sem_in),
                pltpu.async_copy(bs_hbm.at[pl.ds(off, chunk)], bsb.at[buf], sem_in),
            )

        ones = jnp.ones((16,), jnp.float32)
        start(0, 0)
        start(1, 1)

        def wait_bufs(buf):
            pltpu.make_async_copy(yp_hbm.at[pl.ds(0, chunk)], ypb.at[buf], sem_in).wait()
            pltpu.make_async_copy(yt_hbm.at[pl.ds(0, chunk)], ytb.at[buf], sem_in).wait()
            pltpu.make_async_copy(bs_hbm.at[pl.ds(0, chunk)], bsb.at[buf], sem_in).wait()

        def cbody(i, _):
            for buf in range(2):
                c = i * 2 + buf
                wait_bufs(buf)

                @plsc.parallel_loop(0, chunk, step=16, unroll=32)
                def sbody(off):
                    cnt_a, s1_a, s2_a, sr_a = accs
                    b = bsb[buf, pl.ds(off, 16)]
                    t = ytb[buf, pl.ds(off, 16)]
                    p = ypb[buf, pl.ds(off, 16)]
                    d_ = t - p
                    plsc.addupdate_scatter(cnt_a, [b], ones)
                    plsc.addupdate_scatter(s1_a, [b], t)
                    plsc.addupdate_scatter(s2_a, [b], t * t)
                    plsc.addupdate_scatter(sr_a, [b], d_ * d_)

                start(jnp.minimum(c + 2, nchunk - 1), buf)
            return _
        lax.fori_loop(0, nchunk // 2, cbody, None)
        for buf in range(2):
            wait_bufs(buf)

        for st in range(4):
            pltpu.sync_copy(accs[st], shared.at[sid, pl.ds(st * K, K)])
        plsc.subcore_barrier()

        _zero_ref(racc, red_w)
        col = sid * red_w
        prev = pltpu.async_copy(shared.at[0, pl.ds(col, red_w)], rbuf.at[0], sem_red)
        for r in range(NS):
            cur = r % 2
            prev.wait()
            if r + 1 < NS:
                prev = pltpu.async_copy(
                    shared.at[r + 1, pl.ds(col, red_w)], rbuf.at[1 - cur], sem_red)

            @plsc.parallel_loop(0, red_w, step=16, unroll=8)
            def rbody(off):
                racc[pl.ds(off, 16)] += rbuf[cur, pl.ds(off, 16)]

        pltpu.sync_copy(racc, out_hbm.at[cid, pl.ds(col, red_w)])

    return _accumulate


@functools.partial(
    pl.kernel,
    out_type=jax.ShapeDtypeStruct((16,), jnp.float32),
    mesh=_mesh,
    scratch_types=[
        pltpu.VMEM((A4K,), jnp.float32),
        pltpu.VMEM((A4K,), jnp.float32),
        pltpu.VMEM((K,), jnp.int32),
        pltpu.VMEM((256,), jnp.int32),
        pltpu.VMEM((256,), jnp.int32),
        pltpu.VMEM((16,), jnp.float32),
        pltpu.SemaphoreType.DMA,
    ],
    compiler_params=_params,
)
def _finalize(part_hbm, out_hbm, pa, pb, keys, ha, hb, obuf, sem):
    cid = lax.axis_index("c")
    sid = lax.axis_index("s")
    flip = jnp.full((16,), _FLIP, jnp.int32)
    sign = jnp.full((16,), _SIGN, jnp.int32)

    @pl.when(jnp.logical_and(cid == 0, sid == 0))
    def _():
        ca = pltpu.async_copy(part_hbm.at[0], pa, sem)
        cb = pltpu.async_copy(part_hbm.at[1], pb, sem)
        ca.wait()
        cb.wait()

        @plsc.parallel_loop(0, A4K, step=16, unroll=8)
        def add_body(off):
            pa[pl.ds(off, 16)] += pb[pl.ds(off, 16)]

        @plsc.parallel_loop(0, K, step=16, unroll=8, carry=_Z16I())
        def nse_body(off, lcount):
            cnt = pa[pl.ds(off, 16)]
            s1 = pa[pl.ds(K + off, 16)]
            s2 = pa[pl.ds(2 * K + off, 16)]
            sr = pa[pl.ds(3 * K + off, 16)]
            present = cnt > 0.0
            ss_tot = s2 - s1 * s1 / jnp.maximum(cnt, 1.0)
            nse = 1.0 - sr / (ss_tot + 1e-10)
            nse_m = jnp.where(present, nse, jnp.float32(jnp.inf))
            u32 = plsc.bitcast(nse_m, jnp.int32)
            keys[pl.ds(off, 16)] = jnp.where(
                u32 < 0, jnp.bitwise_not(u32), u32 | sign)
            return lcount + plsc.all_reduce_population_count(present)

        lvec = nse_body

        one = jnp.ones((16,), jnp.int32)
        ones_i = one
        r1 = ((lvec - one) >> 1) + one
        r2 = (lvec >> 1) + one
        pb1 = _Z16I()
        pb2 = _Z16I()

        for lvl in range(4):
            sh = 24 - 8 * lvl
            mb = 0 if lvl == 0 else (0xFFFFFFFF << (32 - 8 * lvl)) & 0xFFFFFFFF
            if mb >= 0x80000000:
                mb -= 0x100000000
            maskbits = jnp.full((16,), mb, jnp.int32)
            for j in range(16):
                ha[pl.ds(j * 16, 16)] = _Z16I()
                hb[pl.ds(j * 16, 16)] = _Z16I()

            pb1c, pb2c = pb1, pb2

            @plsc.parallel_loop(0, K, step=16, unroll=8)
            def hist_body(off):
                kv = keys[pl.ds(off, 16)]
                shv = jnp.full((16,), sh, jnp.int32)
                dg = (lax.shift_right_logical(kv, shv) if sh else kv) & 0xFF
                m1 = (kv & maskbits) == pb1c
                m2 = (kv & maskbits) == pb2c
                plsc.addupdate_scatter(ha, [dg], ones_i, mask=m1)
                plsc.addupdate_scatter(hb, [dg], ones_i, mask=m2)

            carry1 = jnp.int32(0)
            carry2 = jnp.int32(0)
            b1 = _Z16I()
            b2 = _Z16I()
            cumb1 = jnp.int32(0)
            cumb2 = jnp.int32(0)
            for j in range(16):
                v1 = ha[pl.ds(j * 16, 16)]
                v2 = hb[pl.ds(j * 16, 16)]
                cum1 = plsc.cumsum(v1) + carry1
                cum2 = plsc.cumsum(v2) + carry2
                less1 = cum1 < r1
                less2 = cum2 < r2
                b1 = b1 + plsc.all_reduce_population_count(less1)
                b2 = b2 + plsc.all_reduce_population_count(less2)
                cumb1 = jnp.maximum(cumb1, jnp.max(jnp.where(less1, cum1, 0)))
                cumb2 = jnp.maximum(cumb2, jnp.max(jnp.where(less2, cum2, 0)))
                carry1 = jnp.max(cum1)
                carry2 = jnp.max(cum2)
            pb1 = pb1 | (b1 << sh)
            pb2 = pb2 | (b2 << sh)
            r1 = r1 - cumb1
            r2 = r2 - cumb2

        k1 = pb1 ^ sign
        k2 = pb2 ^ sign
        f1 = plsc.bitcast(jnp.where(k1 < 0, k1 ^ flip, k1), jnp.float32)
        f2 = plsc.bitcast(jnp.where(k2 < 0, k2 ^ flip, k2), jnp.float32)
        obuf[...] = 0.5 * (f1 + f2)
        pltpu.sync_copy(obuf, out_hbm)


def kernel(y_pred, y_true, basin):
    y_pred = jnp.ravel(y_pred)
    y_true = jnp.ravel(y_true)
    basin = jnp.ravel(basin)
    partials = _build(y_pred.shape[0])(y_pred, y_true, basin)
    return _finalize(partials)[0]


# speedup vs baseline: 1.0186x; 1.0186x over previous
import functools

import jax
import jax.numpy as jnp
from jax import lax
from jax.experimental import pallas as pl
from jax.experimental.pallas import tpu as pltpu
from jax.experimental.pallas import tpu_sc as plsc

K = 4096
NC, NS = 2, 16
NW = NC * NS
A4K = 4 * K
CHUNK = 8192

_mesh = plsc.VectorSubcoreMesh(
    core_axis_name="c", subcore_axis_name="s", num_cores=NC, num_subcores=NS)
_params = pltpu.CompilerParams(needs_layout_passes=False)

_Z16F = functools.partial(jnp.zeros, (16,), jnp.float32)
_Z16I = functools.partial(jnp.zeros, (16,), jnp.int32)
_SIGN = -2147483648
_FLIP = 0x7FFFFFFF


def _zero_ref(ref, nwords, dtype=jnp.float32):
    z = jnp.zeros((16,), dtype)

    @plsc.parallel_loop(0, nwords, step=16, unroll=8)
    def body(off):
        ref[pl.ds(off, 16)] = z


@functools.lru_cache(maxsize=None)
def _build(n):
    assert n % NW == 0
    per_w = n // NW
    chunk = min(CHUNK, per_w)
    assert per_w % chunk == 0 and chunk % 64 == 0
    nchunk = per_w // chunk
    assert nchunk % 2 == 0, "ring pipeline processes chunks in pairs"
    red_w = A4K // NS

    @functools.partial(
        pl.kernel,
        out_type=jax.ShapeDtypeStruct((NC, A4K), jnp.float32),
        mesh=_mesh,
        scratch_types=[
            [pltpu.VMEM((K,), jnp.float32)] * 4,
            pltpu.VMEM((2, chunk), jnp.float32),
            pltpu.VMEM((2, chunk), jnp.float32),
            pltpu.VMEM((2, chunk), jnp.int32),
            pltpu.VMEM_SHARED((NS, A4K), jnp.float32),
            pltpu.VMEM((red_w,), jnp.float32),
            pltpu.VMEM((2, red_w), jnp.float32),
            pltpu.SemaphoreType.DMA,
            pltpu.SemaphoreType.DMA,
        ],
        compiler_params=_params,
    )
    def _accumulate(yp_hbm, yt_hbm, bs_hbm, out_hbm,
                    accs, ypb, ytb, bsb, shared, racc, rbuf, sem_in, sem_red):
        cid = lax.axis_index("c")
        sid = lax.axis_index("s")
        wid = cid * NS + sid
        base = wid * per_w
        for a in accs:
            _zero_ref(a, K)

        def start(c, buf):
            off = base + c * chunk
            return (
                pltpu.async_copy(yp_hbm.at[pl.ds(off, chunk)], ypb.at[buf], sem_in),
                pltpu.async_copy(yt_hbm.at[pl.ds(off, chunk)], ytb.at[buf], sem_in),
                pltpu.async_copy(bs_hbm.at[pl.ds(off, chunk)], bsb.at[buf], sem_in),
            )

        ones = jnp.ones((16,), jnp.float32)
        start(0, 0)
        start(1, 1)

        def wait_bufs(buf):
            pltpu.make_async_copy(yp_hbm.at[pl.ds(0, chunk)], ypb.at[buf], sem_in).wait()
            pltpu.make_async_copy(yt_hbm.at[pl.ds(0, chunk)], ytb.at[buf], sem_in).wait()
            pltpu.make_async_copy(bs_hbm.at[pl.ds(0, chunk)], bsb.at[buf], sem_in).wait()

        def cbody(i, _):
            for buf in range(2):
                c = i * 2 + buf
                wait_bufs(buf)

                @plsc.parallel_loop(0, chunk, step=16, unroll=32)
                def sbody(off):
                    cnt_a, s1_a, s2_a, sr_a = accs
                    b = bsb[buf, pl.ds(off, 16)]
                    t = ytb[buf, pl.ds(off, 16)]
                    p = ypb[buf, pl.ds(off, 16)]
                    d_ = t - p
                    plsc.addupdate_scatter(cnt_a, [b], ones)
                    plsc.addupdate_scatter(s1_a, [b], t)
                    plsc.addupdate_scatter(s2_a, [b], t * t)
                    plsc.addupdate_scatter(sr_a, [b], d_ * d_)

                start(jnp.minimum(c + 2, nchunk - 1), buf)
            return _
        lax.fori_loop(0, nchunk // 2, cbody, None)
        for buf in range(2):
            wait_bufs(buf)

        for st in range(4):
            pltpu.sync_copy(accs[st], shared.at[sid, pl.ds(st * K, K)])
        plsc.subcore_barrier()

        _zero_ref(racc, red_w)
        col = sid * red_w
        prev = pltpu.async_copy(shared.at[0, pl.ds(col, red_w)], rbuf.at[0], sem_red)
        for r in range(NS):
            cur = r % 2
            prev.wait()
            if r + 1 < NS:
                prev = pltpu.async_copy(
                    shared.at[r + 1, pl.ds(col, red_w)], rbuf.at[1 - cur], sem_red)

            @plsc.parallel_loop(0, red_w, step=16, unroll=8)
            def rbody(off):
                racc[pl.ds(off, 16)] += rbuf[cur, pl.ds(off, 16)]

        pltpu.sync_copy(racc, out_hbm.at[cid, pl.ds(col, red_w)])

    return _accumulate


BAS = K // NS
ROW = 2 * BAS


@functools.partial(
    pl.kernel,
    out_type=jax.ShapeDtypeStruct((16,), jnp.float32),
    mesh=_mesh,
    scratch_types=[
        pltpu.VMEM((8, BAS), jnp.float32),
        pltpu.VMEM((ROW,), jnp.int32),
        pltpu.VMEM_SHARED((NS, ROW), jnp.int32),
        pltpu.VMEM((NS, ROW), jnp.int32),
        pltpu.VMEM((K,), jnp.int32),
        pltpu.VMEM((256,), jnp.int32),
        pltpu.VMEM((256,), jnp.int32),
        pltpu.VMEM((16,), jnp.float32),
        pltpu.SemaphoreType.DMA,
    ],
    compiler_params=_params,
)
def _finalize(part_hbm, out_hbm, pt, trow, srows, grows, keys, ha, hb,
              obuf, sem):
    cid = lax.axis_index("c")
    sid = lax.axis_index("s")
    flip = jnp.full((16,), _FLIP, jnp.int32)
    sign = jnp.full((16,), _SIGN, jnp.int32)

    @pl.when(cid == 0)
    def _():
        cps = []
        for c in range(NC):
            for st in range(4):
                cps.append(pltpu.async_copy(
                    part_hbm.at[c, pl.ds(st * K + sid * BAS, BAS)],
                    pt.at[c * 4 + st], sem))
        for d in cps:
            d.wait()

        @plsc.parallel_loop(0, BAS, step=16, unroll=8, carry=_Z16I())
        def nse_body(off, lcount):
            cnt = pt[0, pl.ds(off, 16)] + pt[4, pl.ds(off, 16)]
            s1 = pt[1, pl.ds(off, 16)] + pt[5, pl.ds(off, 16)]
            s2 = pt[2, pl.ds(off, 16)] + pt[6, pl.ds(off, 16)]
            sr = pt[3, pl.ds(off, 16)] + pt[7, pl.ds(off, 16)]
            present = cnt > 0.0
            ss_tot = s2 - s1 * s1 / jnp.maximum(cnt, 1.0)
            nse = 1.0 - sr / (ss_tot + 1e-10)
            nse_m = jnp.where(present, nse, jnp.float32(jnp.inf))
            u32 = plsc.bitcast(nse_m, jnp.int32)
            trow[pl.ds(off, 16)] = jnp.where(
                u32 < 0, jnp.bitwise_not(u32), u32 | sign)
            return lcount + plsc.all_reduce_population_count(present)

        trow[pl.ds(BAS, 16)] = nse_body
        pltpu.sync_copy(trow, srows.at[sid])
        plsc.subcore_barrier()

    @pl.when(jnp.logical_and(cid == 0, sid == 0))
    def _():
        pltpu.sync_copy(srows, grows)
        lvec = _Z16I()

        @plsc.parallel_loop(0, K, step=16, unroll=8)
        def flat_body(off):
            r = lax.div(off, BAS)
            c = lax.rem(off, BAS)
            keys[pl.ds(off, 16)] = grows[r, pl.ds(c, 16)]

        for r in range(NS):
            lvec = lvec + grows[r, pl.ds(BAS, 16)]

        one = jnp.ones((16,), jnp.int32)
        ones_i = one
        r1 = ((lvec - one) >> 1) + one
        r2 = (lvec >> 1) + one
        pb1 = _Z16I()
        pb2 = _Z16I()

        for lvl in range(4):
            sh = 24 - 8 * lvl
            mb = 0 if lvl == 0 else (0xFFFFFFFF << (32 - 8 * lvl)) & 0xFFFFFFFF
            if mb >= 0x80000000:
                mb -= 0x100000000
            maskbits = jnp.full((16,), mb, jnp.int32)
            for j in range(16):
                ha[pl.ds(j * 16, 16)] = _Z16I()
                hb[pl.ds(j * 16, 16)] = _Z16I()

            pb1c, pb2c = pb1, pb2

            @plsc.parallel_loop(0, K, step=16, unroll=8)
            def hist_body(off):
                kv = keys[pl.ds(off, 16)]
                shv = jnp.full((16,), sh, jnp.int32)
                dg = (lax.shift_right_logical(kv, shv) if sh else kv) & 0xFF
                m1 = (kv & maskbits) == pb1c
                m2 = (kv & maskbits) == pb2c
                plsc.addupdate_scatter(ha, [dg], ones_i, mask=m1)
                plsc.addupdate_scatter(hb, [dg], ones_i, mask=m2)

            carry1 = jnp.int32(0)
            carry2 = jnp.int32(0)
            b1 = _Z16I()
            b2 = _Z16I()
            cumb1 = jnp.int32(0)
            cumb2 = jnp.int32(0)
            for j in range(16):
                v1 = ha[pl.ds(j * 16, 16)]
                v2 = hb[pl.ds(j * 16, 16)]
                cum1 = plsc.cumsum(v1) + carry1
                cum2 = plsc.cumsum(v2) + carry2
                less1 = cum1 < r1
                less2 = cum2 < r2
                b1 = b1 + plsc.all_reduce_population_count(less1)
                b2 = b2 + plsc.all_reduce_population_count(less2)
                cumb1 = jnp.maximum(cumb1, jnp.max(jnp.where(less1, cum1, 0)))
                cumb2 = jnp.maximum(cumb2, jnp.max(jnp.where(less2, cum2, 0)))
                carry1 = jnp.max(cum1)
                carry2 = jnp.max(cum2)
            pb1 = pb1 | (b1 << sh)
            pb2 = pb2 | (b2 << sh)
            r1 = r1 - cumb1
            r2 = r2 - cumb2

        k1 = pb1 ^ sign
        k2 = pb2 ^ sign
        f1 = plsc.bitcast(jnp.where(k1 < 0, k1 ^ flip, k1), jnp.float32)
        f2 = plsc.bitcast(jnp.where(k2 < 0, k2 ^ flip, k2), jnp.float32)
        obuf[...] = 0.5 * (f1 + f2)
        pltpu.sync_copy(obuf, out_hbm)


def kernel(y_pred, y_true, basin):
    y_pred = jnp.ravel(y_pred)
    y_true = jnp.ravel(y_true)
    basin = jnp.ravel(basin)
    partials = _build(y_pred.shape[0])(y_pred, y_true, basin)
    return _finalize(partials)[0]
